# Initial kernel scaffold; baseline (speedup 1.0000x reference)
import jax
import jax.numpy as jnp
from jax.experimental import pallas as pl


def kernel(inputs, e_feats, rain0, edge_index, W_emb, W_in0, W_in1, W_in2, W_oe, W_on, w_rain):
    raise NotImplementedError("write your pallas kernel here")


# speedup vs baseline: 1.6732x; 1.6732x over previous
import functools

import jax
import jax.numpy as jnp
from jax import lax
from jax.experimental import pallas as pl
from jax.experimental.pallas import tpu as pltpu
from jax.experimental.pallas import tpu_sc as plsc

N = 10000
E = 160000
IN_DIM = 128
H = 64
EDGE_DIM = 4
NUM_LAYERS = 6
TW = 32

NC, NS = 2, 16
NWORK = NC * NS
CH = 128
NCHUNK = E // CH
NT_G = (NCHUNK + NWORK - 1) // NWORK
NPC = NCHUNK // NC
NT_S = (NPC + NS - 1) // NS
ROWS_PER_TILE = N // NS
ZCH = 125

BN = 1000
BE = 2000

_DOT = dict(precision=lax.Precision.HIGHEST, preferred_element_type=jnp.float32)


def _table_block(h, rain_next, wi0, wi1, wi2):
    wsrc = jnp.concatenate([wi0[0:64], wi1[0:64], wi2[0:64]], axis=1)
    wdst = jnp.concatenate([wi0[64:128], wi1[64:128], wi2[64:128]], axis=1)
    wr = jnp.concatenate([wi0[136:137], wi1[136:137], wi2[136:137]], axis=1)
    ts = jnp.dot(h, wsrc, **_DOT) + rain_next * wr
    td = jnp.dot(h, wdst, **_DOT)
    z = jnp.zeros((h.shape[0], TW - 24), jnp.float32)
    return jnp.concatenate([ts, z], axis=1), jnp.concatenate([td, z], axis=1)


def _emb_body(x_ref, rain_ref, wemb_ref, wi0_ref, wi1_ref, wi2_ref,
              ts_ref, td_ref):
    h = jnp.dot(x_ref[...], wemb_ref[...], **_DOT)
    rain_next = rain_ref[:, 0:1]
    ts, td = _table_block(h, rain_next, wi0_ref[...], wi1_ref[...], wi2_ref[...])
    ts_ref[...] = ts
    td_ref[...] = td


def _emb_tables(inputs, rain0, W_emb, W_in0, W_in1, W_in2):
    return pl.pallas_call(
        _emb_body,
        grid=(N // BN,),
        in_specs=[
            pl.BlockSpec((BN, IN_DIM), lambda i: (i, 0)),
            pl.BlockSpec((BN, NUM_LAYERS), lambda i: (i, 0)),
            pl.BlockSpec((IN_DIM, H), lambda i: (0, 0)),
            pl.BlockSpec((137, 8), lambda i: (0, 0)),
            pl.BlockSpec((137, 8), lambda i: (0, 0)),
            pl.BlockSpec((137, 8), lambda i: (0, 0)),
        ],
        out_specs=(
            pl.BlockSpec((BN, TW), lambda i: (i, 0)),
            pl.BlockSpec((BN, TW), lambda i: (i, 0)),
        ),
        out_shape=(
            jax.ShapeDtypeStruct((N, TW), jnp.float32),
            jax.ShapeDtypeStruct((N, TW), jnp.float32),
        ),
    )(inputs, rain0, W_emb, W_in0, W_in1, W_in2)


def _node_body(layer, with_tables, agg_ref, rain_ref, won_ref, wrain_ref,
               wi0_ref, wi1_ref, wi2_ref, rout_ref, *table_refs):
    agg = agg_ref[0] + agg_ref[1]
    bias = rain_ref[:, layer:layer + 1]
    h = jnp.maximum(jnp.dot(agg, won_ref[...], **_DOT) + bias, 0.0)
    rout_ref[...] = jnp.dot(h, wrain_ref[...], **_DOT)
    if with_tables:
        rain_next = rain_ref[:, layer + 1:layer + 2]
        ts, td = _table_block(h, rain_next,
                              wi0_ref[...], wi1_ref[...], wi2_ref[...])
        table_refs[0][...] = ts
        table_refs[1][...] = td


def _node_update(agg2, rain0, W_on, w_rain, W_in0, W_in1, W_in2, layer,
                 with_tables):
    out_shape = [jax.ShapeDtypeStruct((N, 1), jnp.float32)]
    out_specs = [pl.BlockSpec((BN, 1), lambda i: (i, 0))]
    if with_tables:
        out_shape += [jax.ShapeDtypeStruct((N, TW), jnp.float32)] * 2
        out_specs += [pl.BlockSpec((BN, TW), lambda i: (i, 0))] * 2
    return pl.pallas_call(
        functools.partial(_node_body, layer, with_tables),
        grid=(N // BN,),
        in_specs=[
            pl.BlockSpec((NC, BN, H), lambda i: (0, i, 0)),
            pl.BlockSpec((BN, NUM_LAYERS), lambda i: (i, 0)),
            pl.BlockSpec((H, H), lambda i: (0, 0)),
            pl.BlockSpec((H, 1), lambda i: (0, 0)),
            pl.BlockSpec((137, 8), lambda i: (0, 0)),
            pl.BlockSpec((137, 8), lambda i: (0, 0)),
            pl.BlockSpec((137, 8), lambda i: (0, 0)),
        ],
        out_specs=tuple(out_specs),
        out_shape=tuple(out_shape),
    )(agg2, rain0, W_on, w_rain, W_in0, W_in1, W_in2)


_SC_MESH = plsc.VectorSubcoreMesh(core_axis_name="c", subcore_axis_name="s",
                                  num_cores=NC, num_subcores=NS)


@functools.partial(
    pl.kernel,
    out_type=(jax.ShapeDtypeStruct((E, TW), jnp.float32),
              jax.ShapeDtypeStruct((E, TW), jnp.float32)),
    mesh=_SC_MESH,
    compiler_params=pltpu.CompilerParams(use_tc_tiling_on_sc=False),
    scratch_types=[
        pltpu.VMEM((CH,), jnp.int32),
        pltpu.VMEM((CH,), jnp.int32),
        pltpu.VMEM((CH, TW), jnp.float32),
        pltpu.VMEM((CH, TW), jnp.float32),
        pltpu.SemaphoreType.DMA,
        pltpu.SemaphoreType.DMA,
    ],
)
def _gather_call(src_hbm, dst_hbm, ts_hbm, td_hbm, o1_hbm, o2_hbm,
                 si, di, b1, b2, sem1, sem2):
    wid = lax.axis_index("s") * NC + lax.axis_index("c")

    def step(t, carry):
        c = wid + t * NWORK

        @pl.when(c < NCHUNK)
        def _():
            base = c * CH
            pltpu.sync_copy(src_hbm.at[pl.ds(base, CH)], si)
            pltpu.sync_copy(dst_hbm.at[pl.ds(base, CH)], di)
            cp1 = pltpu.async_copy(ts_hbm.at[si], b1, sem1)
            cp2 = pltpu.async_copy(td_hbm.at[di], b2, sem2)
            cp1.wait()
            cp2.wait()
            pltpu.sync_copy(b1, o1_hbm.at[pl.ds(base, CH)])
            pltpu.sync_copy(b2, o2_hbm.at[pl.ds(base, CH)])

        return carry

    lax.fori_loop(0, NT_G, step, 0)


def _edge_body(layer, s1_ref, s2_ref, e_ref, wi0_ref, wi1_ref, wi2_ref,
               woe_ref, msg_ref):
    del layer
    e = e_ref[0]
    ef = jnp.concatenate([e, e], axis=1)
    s = s1_ref[...] + s2_ref[...]
    for j, wref in enumerate((wi0_ref, wi1_ref, wi2_ref)):
        w = wref[128:136, :]
        ef = jnp.maximum(s[:, 8 * j:8 * j + 8] + jnp.dot(ef, w, **_DOT), 0.0)
    woe = woe_ref[...]
    wsum = woe[8:12] + woe[12:16] + woe[16:20]
    msg_ref[...] = jnp.maximum(
        jnp.dot(ef, woe[0:8], **_DOT) + jnp.dot(e, wsum, **_DOT), 0.0)


def _edge_msgs(s1, s2, eT, W_in0, W_in1, W_in2, W_oe, layer):
    return pl.pallas_call(
        functools.partial(_edge_body, layer),
        grid=(E // BE,),
        in_specs=[
            pl.BlockSpec((BE, TW), lambda i: (i, 0)),
            pl.BlockSpec((BE, TW), lambda i: (i, 0)),
            pl.BlockSpec((1, BE, EDGE_DIM), lambda i, L=layer: (L, i, 0)),
            pl.BlockSpec((137, 8), lambda i: (0, 0)),
            pl.BlockSpec((137, 8), lambda i: (0, 0)),
            pl.BlockSpec((137, 8), lambda i: (0, 0)),
            pl.BlockSpec((20, H), lambda i: (0, 0)),
        ],
        out_specs=pl.BlockSpec((BE, H), lambda i: (i, 0)),
        out_shape=jax.ShapeDtypeStruct((E, H), jnp.float32),
        compiler_params=pltpu.CompilerParams(vmem_limit_bytes=100 * 1024 * 1024),
    )(s1, s2, eT, W_in0, W_in1, W_in2, W_oe)


@functools.partial(
    pl.kernel,
    out_type=jax.ShapeDtypeStruct((NC * N, H), jnp.float32),
    mesh=_SC_MESH,
    compiler_params=pltpu.CompilerParams(use_tc_tiling_on_sc=False),
    scratch_types=[
        pltpu.VMEM((CH,), jnp.int32),
        pltpu.VMEM((CH, H), jnp.float32),
        pltpu.VMEM((ZCH, H), jnp.float32),
        pltpu.VMEM_SHARED((N, H), jnp.float32),
    ],
)
def _scatter_call(dst_hbm, msg_hbm, out_hbm, di, mb, zb, acc):
    cid = lax.axis_index("c")
    sid = lax.axis_index("s")

    def zrow(i, carry):
        for j in range(H // 16):
            zb[i, pl.ds(16 * j, 16)] = jnp.zeros((16,), jnp.float32)
        return carry

    lax.fori_loop(0, ZCH, zrow, 0, unroll=4)
    for z in range(ROWS_PER_TILE // ZCH):
        pltpu.sync_copy(zb, acc.at[pl.ds(sid * ROWS_PER_TILE + z * ZCH, ZCH)])
    plsc.subcore_barrier()

    def step(t, carry):
        c = cid * NPC + sid + t * NS

        @pl.when(c < (cid + 1) * NPC)
        def _():
            base = c * CH
            pltpu.sync_copy(dst_hbm.at[pl.ds(base, CH)], di)
            pltpu.sync_copy(msg_hbm.at[pl.ds(base, CH)], mb)
            pltpu.sync_copy(mb, acc.at[di], add=True)

        return carry

    lax.fori_loop(0, NT_S, step, 0)
    plsc.subcore_barrier()

    pltpu.sync_copy(
        acc.at[pl.ds(sid * ROWS_PER_TILE, ROWS_PER_TILE)],
        out_hbm.at[pl.ds(cid * N + sid * ROWS_PER_TILE, ROWS_PER_TILE)])


def kernel(inputs, e_feats, rain0, edge_index, W_emb, W_in0, W_in1, W_in2,
           W_oe, W_on, w_rain):
    src = edge_index[0].astype(jnp.int32)
    dst = edge_index[1].astype(jnp.int32)
    eT = jnp.transpose(e_feats, (2, 0, 1))

    ts, td = _emb_tables(inputs, rain0, W_emb, W_in0, W_in1, W_in2)
    rains = []
    for l in range(NUM_LAYERS):
        s1, s2 = _gather_call(src, dst, ts, td)
        msg = _edge_msgs(s1, s2, eT, W_in0, W_in1, W_in2, W_oe, l)
        agg = _scatter_call(dst, msg).reshape(NC, N, H)
        if l < NUM_LAYERS - 1:
            rout, ts, td = _node_update(agg, rain0, W_on, w_rain,
                                        W_in0, W_in1, W_in2, l, True)
        else:
            (rout,) = _node_update(agg, rain0, W_on, w_rain,
                                   W_in0, W_in1, W_in2, l, False)
        rains.append(rout)
    return jnp.concatenate(rains, axis=1)
